# Initial kernel scaffold; baseline (speedup 1.0000x reference)
import jax
import jax.numpy as jnp
from jax.experimental import pallas as pl


def kernel(x, edge_index, edge_weight, W_z, b_z, Wl_z, bl_z, W_r, b_r, Wl_r, bl_r, W_h, b_h, Wl_h, bl_h, att, W_out, b_out):
    raise NotImplementedError("write your pallas kernel here")


# speedup vs baseline: 50.4334x; 50.4334x over previous
import functools

import jax
import jax.numpy as jnp
from jax import lax
from jax.experimental import pallas as pl
from jax.experimental.pallas import tpu as pltpu
from jax.experimental.pallas import tpu_sc as plsc

B = 128
N = 2000
E = 16000
F = 24
HID = 16
P = 12
FOUT = 12

NPAD = 2064
XW = N * F
XSL = XW + 16
ECH = 8000
NWORK = 32
BPW = B // NWORK
DSL = 2048 // NWORK


def _sc_body(x_hbm, src_hbm, dst_hbm, ew_hbm, agg_hbm, dinv_hbm,
             deg_v, ew_v, src_v, dst_v, x_v, agg_v):
    wid = lax.axis_index("s") * 2 + lax.axis_index("c")

    pltpu.sync_copy(ew_hbm, ew_v)

    def _zero_deg(i, carry):
        deg_v[pl.ds(i * 16, 16)] = jnp.zeros((16,), jnp.float32)
        return carry
    lax.fori_loop(0, NPAD // 16, _zero_deg, 0)

    lanes = lax.iota(jnp.int32, 16)
    lane0 = jnp.where(lanes == 0, jnp.float32(1.0), jnp.float32(0.0))

    for c in range(E // ECH):
        pltpu.sync_copy(dst_hbm.at[pl.ds(c * ECH, ECH)], dst_v)

        def _deg(g, carry):
            d16 = dst_v[pl.ds(g * 16, 16)]
            w16 = ew_v[pl.ds(c * ECH + g * 16, 16)]
            for j in range(16):
                plsc.addupdate(deg_v.at[pl.ds(d16[j], 16)], w16[j] * lane0)
            return carry
        lax.fori_loop(0, ECH // 16, _deg, 0)

    def _nr(i, carry):
        dv = deg_v[pl.ds(i * 16, 16)] + 1.0
        bi = lax.bitcast_convert_type(dv, jnp.int32)
        bi = jnp.int32(0x5F3759DF) - lax.shift_right_arithmetic(bi, 1)
        y = lax.bitcast_convert_type(bi, jnp.float32)
        y = y * (1.5 - 0.5 * dv * y * y)
        y = y * (1.5 - 0.5 * dv * y * y)
        y = y * (1.5 - 0.5 * dv * y * y)
        y = y * (1.5 - 0.5 * dv * y * y)
        deg_v[pl.ds(i * 16, 16)] = y
        return carry
    lax.fori_loop(0, NPAD // 16, _nr, 0)

    pltpu.sync_copy(deg_v.at[pl.ds(wid * DSL, DSL)],
                    dinv_hbm.at[pl.ds(wid * DSL, DSL)])

    for c in range(E // ECH):
        pltpu.sync_copy(src_hbm.at[pl.ds(c * ECH, ECH)], src_v)
        pltpu.sync_copy(dst_hbm.at[pl.ds(c * ECH, ECH)], dst_v)

        def _norm(i, carry):
            s16 = src_v[pl.ds(i * 16, 16)]
            d16 = dst_v[pl.ds(i * 16, 16)]
            gs = plsc.load_gather(deg_v, [s16])
            gd = plsc.load_gather(deg_v, [d16])
            off = c * ECH + i * 16
            ew_v[pl.ds(off, 16)] = ew_v[pl.ds(off, 16)] * gs * gd
            return carry
        lax.fori_loop(0, ECH // 16, _norm, 0)

    tailmask = jnp.where(lanes < F - 16, jnp.float32(1.0), jnp.float32(0.0))
    x_v[pl.ds(XW, 16)] = jnp.zeros((16,), jnp.float32)

    for bl in range(BPW):
        b = wid * BPW + bl
        pltpu.sync_copy(x_hbm.at[pl.ds(b * XW, XW)], x_v.at[pl.ds(0, XW)])

        def _zero_agg(i, carry):
            agg_v[pl.ds(i * 16, 16)] = jnp.zeros((16,), jnp.float32)
            return carry
        lax.fori_loop(0, XSL // 16, _zero_agg, 0)

        for c in range(E // ECH):
            pltpu.sync_copy(src_hbm.at[pl.ds(c * ECH, ECH)], src_v)
            pltpu.sync_copy(dst_hbm.at[pl.ds(c * ECH, ECH)], dst_v)

            def _edge(g, carry):
                s16 = src_v[pl.ds(g * 16, 16)] * F
                d16 = dst_v[pl.ds(g * 16, 16)] * F
                w16 = ew_v[pl.ds(c * ECH + g * 16, 16)]
                for j in range(16):
                    sb = s16[j]
                    db = d16[j]
                    wv = jnp.full((16,), w16[j], jnp.float32)
                    v0 = x_v[pl.ds(sb, 16)]
                    v1 = x_v[pl.ds(sb + 16, 16)]
                    plsc.addupdate(agg_v.at[pl.ds(db, 16)], v0 * wv)
                    plsc.addupdate(agg_v.at[pl.ds(db + 16, 16)],
                                   v1 * (wv * tailmask))
                return carry
            lax.fori_loop(0, ECH // 16, _edge, 0)

        pltpu.sync_copy(agg_v.at[pl.ds(0, XW)], agg_hbm.at[pl.ds(b * XW, XW)])


_sc_aggregate = functools.partial(
    pl.kernel,
    out_type=[
        jax.ShapeDtypeStruct((B * XW,), jnp.float32),
        jax.ShapeDtypeStruct((2048,), jnp.float32),
    ],
    mesh=plsc.VectorSubcoreMesh(core_axis_name="c", subcore_axis_name="s"),
    compiler_params=pltpu.CompilerParams(needs_layout_passes=False),
    scratch_types=[
        pltpu.VMEM((NPAD,), jnp.float32),
        pltpu.VMEM((E,), jnp.float32),
        pltpu.VMEM((ECH,), jnp.int32),
        pltpu.VMEM((ECH,), jnp.int32),
        pltpu.VMEM((XSL,), jnp.float32),
        pltpu.VMEM((XSL,), jnp.float32),
    ],
)(_sc_body)


def _tc_body(aggT_ref, xT_ref, dinv_ref,
             Wl_zT_ref, W_zT_ref, bl_zT_ref, b_zT_ref,
             Wl_hT_ref, W_hT_ref, bl_hT_ref, b_hT_ref,
             att_ref, W_out_ref, b_out_ref, out_ref):
    dinv = dinv_ref[...]
    dinv2 = dinv * dinv
    Y = aggT_ref[0] + dinv2 * xT_ref[0]

    Wl_zT = Wl_zT_ref[...]
    Wl_hT = Wl_hT_ref[...]
    WzT = jnp.dot(Wl_zT[:, :HID], W_zT_ref[...],
                  preferred_element_type=jnp.float32)
    WhT = jnp.dot(Wl_hT[:, :HID], W_hT_ref[...],
                  preferred_element_type=jnp.float32)
    bzT = bl_zT_ref[...] + jnp.dot(Wl_zT[:, :HID], b_zT_ref[...],
                                   preferred_element_type=jnp.float32)
    bhT = bl_hT_ref[...] + jnp.dot(Wl_hT[:, :HID], b_hT_ref[...],
                                   preferred_element_type=jnp.float32)

    att = att_ref[...]
    ea = jnp.exp(att - jnp.max(att))
    probs = ea / jnp.sum(ea)

    Hacc = jnp.zeros((HID, N), jnp.float32)
    for p in range(P):
        y0 = Y[p:p + 1, :]
        y1 = Y[P + p:P + p + 1, :]
        zpre = WzT[:, 0:1] * y0 + WzT[:, 1:2] * y1 + bzT
        hpre = WhT[:, 0:1] * y0 + WhT[:, 1:2] * y1 + bhT
        Z = jax.nn.sigmoid(zpre)
        Ht = jnp.tanh(hpre)
        Hacc = Hacc + probs[0:1, p:p + 1] * (1.0 - Z) * Ht

    Hr = jnp.maximum(Hacc, 0.0)
    out = lax.dot_general(Hr, W_out_ref[...], (((0,), (0,)), ((), ())),
                          preferred_element_type=jnp.float32)
    out_ref[0] = out + b_out_ref[...]


def kernel(x, edge_index, edge_weight, W_z, b_z, Wl_z, bl_z, W_r, b_r,
           Wl_r, bl_r, W_h, b_h, Wl_h, bl_h, att, W_out, b_out):
    x24 = x.reshape(B, N, F)
    src = edge_index[0]
    dst = edge_index[1]

    agg_flat, dinv = _sc_aggregate(
        x24.reshape(B * XW), src, dst, edge_weight)

    aggT = jnp.transpose(agg_flat.reshape(B, N, F), (0, 2, 1))
    xT = jnp.transpose(x24, (0, 2, 1))
    dinv2d = dinv[:N].reshape(1, N)

    wspec = lambda shape: pl.BlockSpec(shape, lambda b: tuple(0 for _ in shape))
    out = pl.pallas_call(
        _tc_body,
        grid=(B,),
        in_specs=[
            pl.BlockSpec((1, F, N), lambda b: (b, 0, 0)),
            pl.BlockSpec((1, F, N), lambda b: (b, 0, 0)),
            wspec((1, N)),
            wspec((HID, 2 * HID)),
            wspec((HID, 2)),
            wspec((HID, 1)),
            wspec((HID, 1)),
            wspec((HID, 2 * HID)),
            wspec((HID, 2)),
            wspec((HID, 1)),
            wspec((HID, 1)),
            wspec((1, P)),
            wspec((HID, FOUT)),
            wspec((1, FOUT)),
        ],
        out_specs=pl.BlockSpec((1, N, FOUT), lambda b: (b, 0, 0)),
        out_shape=jax.ShapeDtypeStruct((B, N, FOUT), jnp.float32),
    )(
        aggT, xT, dinv2d,
        Wl_z.T, W_z.T, bl_z.reshape(HID, 1), b_z.reshape(HID, 1),
        Wl_h.T, W_h.T, bl_h.reshape(HID, 1), b_h.reshape(HID, 1),
        att.reshape(1, P), W_out, b_out.reshape(1, FOUT),
    )
    return out
